# initial kernel scaffold (unmeasured)
import jax
import jax.numpy as jnp
from jax import lax
from jax.experimental import pallas as pl
from jax.experimental.pallas import tpu as pltpu

Z = 4
M = 4096
D = 4096
CH = M // Z
T = 512
TPC = CH // T
EPS = 1e-6
N_STEP = Z - 1


def kernel(partial, resid, gamma):
    def body(partial_ref, resid_ref, gamma_ref, out_ref,
             recv_hbm, sum_hbm,
             rs_send, rs_recv, ag_send, ag_recv,
             local_sems, tile_a, tile_b, tile_r, tile_o):
        my_x = lax.axis_index("x")
        my_y = lax.axis_index("y")
        my_z = lax.axis_index("z")
        right = (my_x, my_y, (my_z + 1) % Z)
        left = (my_x, my_y, (my_z + Z - 1) % Z)

        barrier = pltpu.get_barrier_semaphore()
        for nbr in (left, right):
            pl.semaphore_signal(barrier, inc=1, device_id=nbr,
                                device_id_type=pl.DeviceIdType.MESH)
        pl.semaphore_wait(barrier, 2)

        def load2(src1, dst1, src2, dst2):
            c1 = pltpu.make_async_copy(src1, dst1, local_sems.at[0])
            c2 = pltpu.make_async_copy(src2, dst2, local_sems.at[1])
            c1.start()
            c2.start()
            c1.wait()
            c2.wait()

        def store(src, dst):
            c = pltpu.make_async_copy(src, dst, local_sems.at[2])
            c.start()
            c.wait()

        g = gamma_ref[...]

        for s in range(N_STEP):
            if s == 0:
                src = partial_ref.at[0, pl.ds((my_z % Z) * CH, CH), :]
            else:
                src = sum_hbm.at[s - 1]
            rdma = pltpu.make_async_remote_copy(
                src_ref=src,
                dst_ref=recv_hbm.at[s],
                send_sem=rs_send.at[s],
                recv_sem=rs_recv.at[s],
                device_id=right,
                device_id_type=pl.DeviceIdType.MESH,
            )
            rdma.start()
            rdma.wait()

            rc = (my_z - s - 1) % Z
            for t in range(TPC):
                load2(recv_hbm.at[s, pl.ds(t * T, T), :], tile_a,
                      partial_ref.at[0, pl.ds(rc * CH + t * T, T), :], tile_b)
                if s < N_STEP - 1:
                    tile_o[...] = tile_a[...] + tile_b[...]
                    store(tile_o, sum_hbm.at[s, pl.ds(t * T, T), :])
                else:
                    rcopy = pltpu.make_async_copy(
                        resid_ref.at[pl.ds(rc * CH + t * T, T), :],
                        tile_r, local_sems.at[2])
                    rcopy.start()
                    rcopy.wait()
                    y = tile_a[...] + tile_b[...] + tile_r[...]
                    rms = jnp.sqrt(
                        jnp.mean(y * y, axis=-1, keepdims=True) + EPS)
                    tile_o[...] = y / rms * g[None, :]
                    store(tile_o, out_ref.at[pl.ds(rc * CH + t * T, T), :])

        q = (my_z + 1) % Z
        for s in range(N_STEP):
            sc = (q - s) % Z
            rdma = pltpu.make_async_remote_copy(
                src_ref=out_ref.at[pl.ds(sc * CH, CH), :],
                dst_ref=out_ref.at[pl.ds(sc * CH, CH), :],
                send_sem=ag_send.at[s],
                recv_sem=ag_recv.at[s],
                device_id=right,
                device_id_type=pl.DeviceIdType.MESH,
            )
            rdma.start()
            rdma.wait()

    return pl.pallas_call(
        body,
        out_shape=jax.ShapeDtypeStruct((M, D), jnp.float32),
        in_specs=[
            pl.BlockSpec(memory_space=pl.ANY),
            pl.BlockSpec(memory_space=pl.ANY),
            pl.BlockSpec(memory_space=pltpu.VMEM),
        ],
        out_specs=pl.BlockSpec(memory_space=pl.ANY),
        scratch_shapes=[
            pltpu.MemorySpace.HBM((N_STEP, CH, D), jnp.float32),
            pltpu.MemorySpace.HBM((N_STEP - 1, CH, D), jnp.float32),
            pltpu.SemaphoreType.DMA((N_STEP,)),
            pltpu.SemaphoreType.DMA((N_STEP,)),
            pltpu.SemaphoreType.DMA((N_STEP,)),
            pltpu.SemaphoreType.DMA((N_STEP,)),
            pltpu.SemaphoreType.DMA((3,)),
            pltpu.VMEM((T, D), jnp.float32),
            pltpu.VMEM((T, D), jnp.float32),
            pltpu.VMEM((T, D), jnp.float32),
            pltpu.VMEM((T, D), jnp.float32),
        ],
        compiler_params=pltpu.CompilerParams(collective_id=0),
    )(partial, resid, gamma)


# baseline (device time: 1222947 ns/iter reference)
import jax
import jax.numpy as jnp
from jax import lax
from jax.experimental import pallas as pl
from jax.experimental.pallas import tpu as pltpu

Z = 4
M = 4096
D = 4096
CH = M // Z
T = 256
TPC = CH // T
EPS = 1e-6
N_STEP = Z - 1


def kernel(partial, resid, gamma):
    def body(partial_ref, resid_ref, gamma_ref,
             out_ref, recv_hbm, sum_hbm,
             rs_send, rs_recv, ag_send, ag_recv,
             local_sems, tile_a, tile_b, tile_r, tile_o):
        my_x = lax.axis_index("x")
        my_y = lax.axis_index("y")
        my_z = lax.axis_index("z")
        right = (my_x, my_y, (my_z + 1) % Z)
        left = (my_x, my_y, (my_z + Z - 1) % Z)

        barrier = pltpu.get_barrier_semaphore()
        for nbr in (left, right):
            pl.semaphore_signal(barrier, inc=1, device_id=nbr,
                                device_id_type=pl.DeviceIdType.MESH)
        pl.semaphore_wait(barrier, 2)

        def load2(src1, dst1, src2, dst2):
            c1 = pltpu.make_async_copy(src1, dst1, local_sems.at[0])
            c2 = pltpu.make_async_copy(src2, dst2, local_sems.at[1])
            c1.start()
            c2.start()
            c1.wait()
            c2.wait()

        def store(src, dst):
            c = pltpu.make_async_copy(src, dst, local_sems.at[2])
            c.start()
            c.wait()

        g = gamma_ref[...]

        for s in range(N_STEP):
            if s == 0:
                src = partial_ref.at[0, pl.ds((my_z % Z) * CH, CH), :]
            else:
                src = sum_hbm.at[s - 1]
            rdma = pltpu.make_async_remote_copy(
                src_ref=src,
                dst_ref=recv_hbm.at[s],
                send_sem=rs_send.at[s],
                recv_sem=rs_recv.at[s],
                device_id=right,
                device_id_type=pl.DeviceIdType.MESH,
            )
            rdma.start()
            rdma.wait()

            rc = (my_z - s - 1) % Z
            for t in range(TPC):
                load2(recv_hbm.at[s, pl.ds(t * T, T), :], tile_a,
                      partial_ref.at[0, pl.ds(rc * CH + t * T, T), :], tile_b)
                if s < N_STEP - 1:
                    tile_o[...] = tile_a[...] + tile_b[...]
                    store(tile_o, sum_hbm.at[s, pl.ds(t * T, T), :])
                else:
                    rcopy = pltpu.make_async_copy(
                        resid_ref.at[pl.ds(rc * CH + t * T, T), :],
                        tile_r, local_sems.at[2])
                    rcopy.start()
                    rcopy.wait()
                    y = tile_a[...] + tile_b[...] + tile_r[...]
                    rms = jnp.sqrt(
                        jnp.mean(y * y, axis=-1, keepdims=True) + EPS)
                    tile_o[...] = y / rms * g[None, :]
                    store(tile_o, out_ref.at[pl.ds(rc * CH + t * T, T), :])

        q = (my_z + 1) % Z
        for s in range(N_STEP):
            sc = (q - s) % Z
            rdma = pltpu.make_async_remote_copy(
                src_ref=out_ref.at[pl.ds(sc * CH, CH), :],
                dst_ref=out_ref.at[pl.ds(sc * CH, CH), :],
                send_sem=ag_send.at[s],
                recv_sem=ag_recv.at[s],
                device_id=right,
                device_id_type=pl.DeviceIdType.MESH,
            )
            rdma.start()
            rdma.wait()

    out, _, _ = pl.pallas_call(
        body,
        out_shape=(
            jax.ShapeDtypeStruct((M, D), jnp.float32),
            jax.ShapeDtypeStruct((N_STEP, CH, D), jnp.float32),
            jax.ShapeDtypeStruct((N_STEP - 1, CH, D), jnp.float32),
        ),
        in_specs=[
            pl.BlockSpec(memory_space=pl.ANY),
            pl.BlockSpec(memory_space=pl.ANY),
            pl.BlockSpec(memory_space=pltpu.VMEM),
        ],
        out_specs=(
            pl.BlockSpec(memory_space=pl.ANY),
            pl.BlockSpec(memory_space=pl.ANY),
            pl.BlockSpec(memory_space=pl.ANY),
        ),
        scratch_shapes=[
            pltpu.SemaphoreType.DMA((N_STEP,)),
            pltpu.SemaphoreType.DMA((N_STEP,)),
            pltpu.SemaphoreType.DMA((N_STEP,)),
            pltpu.SemaphoreType.DMA((N_STEP,)),
            pltpu.SemaphoreType.DMA((3,)),
            pltpu.VMEM((T, D), jnp.float32),
            pltpu.VMEM((T, D), jnp.float32),
            pltpu.VMEM((T, D), jnp.float32),
            pltpu.VMEM((T, D), jnp.float32),
        ],
        compiler_params=pltpu.CompilerParams(collective_id=0),
    )(partial, resid, gamma)
    return out


# device time: 642774 ns/iter; 1.9026x vs baseline; 1.9026x over previous
import jax
import jax.numpy as jnp
from jax import lax
from jax.experimental import pallas as pl
from jax.experimental.pallas import tpu as pltpu

Z = 4
R = 4
M = 4096
D = 4096
Q = M // R
SQ = Q // Z
H = Q // 2
EPS = 1e-6


def _ring_coords(r):
    xr = r // 2
    return xr, (xr + r % 2) % 2


def kernel(partial, resid, gamma):
    def body(partial_ref, resid_ref, gamma_ref,
             out_ref, rs_hbm,
             rs_send, rs_recv, zag_send, zag_recv,
             cw_send, cw_recv, ccw_send, ccw_recv,
             local_sems, t_a, t_b, t_acc, t_o):
        my_x = lax.axis_index("x")
        my_y = lax.axis_index("y")
        my_z = lax.axis_index("z")
        ri = 2 * my_x + (my_x + my_y) % 2
        cw_dev = _ring_coords((ri + 1) % R) + (my_z,)
        ccw_dev = _ring_coords((ri + 3) % R) + (my_z,)
        row0 = ri * Q + my_z * SQ

        barrier = pltpu.get_barrier_semaphore()
        for d in range(1, Z):
            pl.semaphore_signal(
                barrier, inc=1, device_id=(my_x, my_y, (my_z + d) % Z),
                device_id_type=pl.DeviceIdType.MESH)
        for nbr in (cw_dev, ccw_dev):
            pl.semaphore_signal(barrier, inc=1, device_id=nbr,
                                device_id_type=pl.DeviceIdType.MESH)
        pl.semaphore_wait(barrier, 5)

        started = []

        for d in range(1, Z):
            zt = (my_z + d) % Z
            rdma = pltpu.make_async_remote_copy(
                src_ref=partial_ref.at[0, pl.ds(ri * Q + zt * SQ, SQ), :],
                dst_ref=rs_hbm.at[my_z],
                send_sem=rs_send.at[d - 1],
                recv_sem=rs_recv.at[my_z],
                device_id=(my_x, my_y, zt),
                device_id_type=pl.DeviceIdType.MESH,
            )
            rdma.start()
            started.append(rdma)
        for d in range(1, Z):
            zs = (my_z + d) % Z
            recv = pltpu.make_async_remote_copy(
                src_ref=partial_ref.at[0, pl.ds(0, SQ), :],
                dst_ref=rs_hbm.at[zs],
                send_sem=rs_send.at[d - 1],
                recv_sem=rs_recv.at[zs],
                device_id=(my_x, my_y, zs),
                device_id_type=pl.DeviceIdType.MESH,
            )
            recv.wait_recv()

        def load(src, dst, i):
            c = pltpu.make_async_copy(src, dst, local_sems.at[i])
            c.start()
            return c

        z1, z2, z3 = ((my_z + d) % Z for d in range(1, Z))
        c1 = load(partial_ref.at[0, pl.ds(row0, SQ), :], t_a, 0)
        c2 = load(rs_hbm.at[z1], t_b, 1)
        c1.wait()
        c2.wait()
        t_acc[...] = t_a[...] + t_b[...]
        c1 = load(rs_hbm.at[z2], t_a, 0)
        c2 = load(rs_hbm.at[z3], t_b, 1)
        c1.wait()
        c2.wait()
        t_acc[...] = t_acc[...] + t_a[...] + t_b[...]
        c1 = load(resid_ref.at[pl.ds(row0, SQ), :], t_a, 0)
        c1.wait()
        y = t_acc[...] + t_a[...]
        rms = jnp.sqrt(jnp.mean(y * y, axis=-1, keepdims=True) + EPS)
        t_o[...] = y / rms * gamma_ref[...][None, :]
        c1 = load(t_o, out_ref.at[pl.ds(row0, SQ), :], 0)
        c1.wait()

        for d in range(1, Z):
            zt = (my_z + d) % Z
            rdma = pltpu.make_async_remote_copy(
                src_ref=out_ref.at[pl.ds(row0, SQ), :],
                dst_ref=out_ref.at[pl.ds(row0, SQ), :],
                send_sem=zag_send.at[d - 1],
                recv_sem=zag_recv.at[my_z],
                device_id=(my_x, my_y, zt),
                device_id_type=pl.DeviceIdType.MESH,
            )
            rdma.start()
            started.append(rdma)
        for d in range(1, Z):
            zs = (my_z + d) % Z
            recv = pltpu.make_async_remote_copy(
                src_ref=out_ref.at[pl.ds(row0, SQ), :],
                dst_ref=out_ref.at[pl.ds(ri * Q + zs * SQ, SQ), :],
                send_sem=zag_send.at[d - 1],
                recv_sem=zag_recv.at[zs],
                device_id=(my_x, my_y, zs),
                device_id_type=pl.DeviceIdType.MESH,
            )
            recv.wait_recv()

        for s in range(R - 1):
            qc = (ri - s) % R
            qa = (ri + s) % R
            cw = pltpu.make_async_remote_copy(
                src_ref=out_ref.at[pl.ds(qc * Q, H), :],
                dst_ref=out_ref.at[pl.ds(qc * Q, H), :],
                send_sem=cw_send.at[s],
                recv_sem=cw_recv.at[s],
                device_id=cw_dev,
                device_id_type=pl.DeviceIdType.MESH,
            )
            ccw = pltpu.make_async_remote_copy(
                src_ref=out_ref.at[pl.ds(qa * Q + H, H), :],
                dst_ref=out_ref.at[pl.ds(qa * Q + H, H), :],
                send_sem=ccw_send.at[s],
                recv_sem=ccw_recv.at[s],
                device_id=ccw_dev,
                device_id_type=pl.DeviceIdType.MESH,
            )
            cw.start()
            ccw.start()
            started.append(cw)
            started.append(ccw)
            cw.wait_recv()
            ccw.wait_recv()

        for rdma in started:
            rdma.wait_send()

    out, _ = pl.pallas_call(
        body,
        out_shape=(
            jax.ShapeDtypeStruct((M, D), jnp.float32),
            jax.ShapeDtypeStruct((Z, SQ, D), jnp.float32),
        ),
        in_specs=[
            pl.BlockSpec(memory_space=pl.ANY),
            pl.BlockSpec(memory_space=pl.ANY),
            pl.BlockSpec(memory_space=pltpu.VMEM),
        ],
        out_specs=(
            pl.BlockSpec(memory_space=pl.ANY),
            pl.BlockSpec(memory_space=pl.ANY),
        ),
        scratch_shapes=[
            pltpu.SemaphoreType.DMA((Z - 1,)),
            pltpu.SemaphoreType.DMA((Z,)),
            pltpu.SemaphoreType.DMA((Z - 1,)),
            pltpu.SemaphoreType.DMA((Z,)),
            pltpu.SemaphoreType.DMA((R - 1,)),
            pltpu.SemaphoreType.DMA((R - 1,)),
            pltpu.SemaphoreType.DMA((R - 1,)),
            pltpu.SemaphoreType.DMA((R - 1,)),
            pltpu.SemaphoreType.DMA((2,)),
            pltpu.VMEM((SQ, D), jnp.float32),
            pltpu.VMEM((SQ, D), jnp.float32),
            pltpu.VMEM((SQ, D), jnp.float32),
            pltpu.VMEM((SQ, D), jnp.float32),
        ],
        compiler_params=pltpu.CompilerParams(collective_id=0),
    )(partial, resid, gamma)
    return out


# device time: 639272 ns/iter; 1.9130x vs baseline; 1.0055x over previous
import jax
import jax.numpy as jnp
from jax import lax
from jax.experimental import pallas as pl
from jax.experimental.pallas import tpu as pltpu

Z = 4
R = 4
M = 4096
D = 4096
Q = M // R
SQ = Q // Z
EPS = 1e-6
N_HOP = R - 1


def _ring_coords(r):
    xr = r // 2
    return xr, (xr + r % 2) % 2


def kernel(partial, resid, gamma):
    def body(partial_ref, resid_ref, gamma_ref,
             out_ref, rs_hbm,
             rs_send, rs_recv, zag_send, zag_recv,
             cw_send, cw_recv, ccw_send, ccw_recv,
             local_sems, t_a, t_b, t_acc, t_o):
        my_x = lax.axis_index("x")
        my_y = lax.axis_index("y")
        my_z = lax.axis_index("z")
        ri = 2 * my_x + (my_x + my_y) % 2
        cw_dev = _ring_coords((ri + 1) % R) + (my_z,)
        ccw_dev = _ring_coords((ri + 3) % R) + (my_z,)
        row0 = ri * Q + my_z * SQ

        barrier = pltpu.get_barrier_semaphore()
        for d in range(1, Z):
            pl.semaphore_signal(
                barrier, inc=1, device_id=(my_x, my_y, (my_z + d) % Z),
                device_id_type=pl.DeviceIdType.MESH)
        for nbr in (cw_dev, ccw_dev):
            pl.semaphore_signal(barrier, inc=1, device_id=nbr,
                                device_id_type=pl.DeviceIdType.MESH)
        pl.semaphore_wait(barrier, 5)

        started = []

        for d in range(1, Z):
            zt = (my_z + d) % Z
            rdma = pltpu.make_async_remote_copy(
                src_ref=partial_ref.at[0, pl.ds(ri * Q + zt * SQ, SQ), :],
                dst_ref=rs_hbm.at[my_z],
                send_sem=rs_send.at[d - 1],
                recv_sem=rs_recv.at[my_z],
                device_id=(my_x, my_y, zt),
                device_id_type=pl.DeviceIdType.MESH,
            )
            rdma.start()
            started.append(rdma)

        def load(src, dst, i):
            c = pltpu.make_async_copy(src, dst, local_sems.at[i])
            c.start()
            return c

        c1 = load(partial_ref.at[0, pl.ds(row0, SQ), :], t_a, 0)
        c2 = load(resid_ref.at[pl.ds(row0, SQ), :], t_b, 1)
        c1.wait()
        c2.wait()
        t_acc[...] = t_a[...] + t_b[...]
        for d in range(1, Z):
            zs = (my_z + d) % Z
            recv = pltpu.make_async_remote_copy(
                src_ref=partial_ref.at[0, pl.ds(0, SQ), :],
                dst_ref=rs_hbm.at[zs],
                send_sem=rs_send.at[d - 1],
                recv_sem=rs_recv.at[zs],
                device_id=(my_x, my_y, zs),
                device_id_type=pl.DeviceIdType.MESH,
            )
            recv.wait_recv()
            c1 = load(rs_hbm.at[zs], t_a, 0)
            c1.wait()
            t_acc[...] = t_acc[...] + t_a[...]
        y = t_acc[...]
        rms = jnp.sqrt(jnp.mean(y * y, axis=-1, keepdims=True) + EPS)
        t_o[...] = y / rms * gamma_ref[...][None, :]
        c1 = load(t_o, out_ref.at[pl.ds(row0, SQ), :], 0)
        c1.wait()

        for d in range(1, Z):
            zt = (my_z + d) % Z
            rdma = pltpu.make_async_remote_copy(
                src_ref=out_ref.at[pl.ds(row0, SQ), :],
                dst_ref=out_ref.at[pl.ds(row0, SQ), :],
                send_sem=zag_send.at[d - 1],
                recv_sem=zag_recv.at[my_z],
                device_id=(my_x, my_y, zt),
                device_id_type=pl.DeviceIdType.MESH,
            )
            rdma.start()
            started.append(rdma)

        def xy_send(piece_j, quarter, hop, k, sems_s, sems_r, dev):
            rows = quarter * Q + piece_j * SQ
            rdma = pltpu.make_async_remote_copy(
                src_ref=out_ref.at[pl.ds(rows, SQ), :],
                dst_ref=out_ref.at[pl.ds(rows, SQ), :],
                send_sem=sems_s.at[hop, k],
                recv_sem=sems_r.at[hop, k],
                device_id=dev,
                device_id_type=pl.DeviceIdType.MESH,
            )
            rdma.start()
            started.append(rdma)
            return rdma

        def zag_wait(zs):
            recv = pltpu.make_async_remote_copy(
                src_ref=out_ref.at[pl.ds(row0, SQ), :],
                dst_ref=out_ref.at[pl.ds(ri * Q + zs * SQ, SQ), :],
                send_sem=zag_send.at[0],
                recv_sem=zag_recv.at[zs],
                device_id=(my_x, my_y, zs),
                device_id_type=pl.DeviceIdType.MESH,
            )
            recv.wait_recv()

        def xy_wait(piece_j, quarter, hop, k, sems_s, sems_r, dev):
            recv = pltpu.make_async_remote_copy(
                src_ref=out_ref.at[pl.ds(row0, SQ), :],
                dst_ref=out_ref.at[pl.ds(quarter * Q + piece_j * SQ, SQ), :],
                send_sem=sems_s.at[hop, k],
                recv_sem=sems_r.at[hop, k],
                device_id=dev,
                device_id_type=pl.DeviceIdType.MESH,
            )
            recv.wait_recv()

        cw_j = [my_z, (my_z + 1) % Z]
        ccw_j = [(my_z + 2) % Z, (my_z + 3) % Z]

        xy_send(cw_j[0], ri, 0, 0, cw_send, cw_recv, cw_dev)
        zag_wait(cw_j[1])
        xy_send(cw_j[1], ri, 0, 1, cw_send, cw_recv, cw_dev)
        zag_wait(ccw_j[0])
        xy_send(ccw_j[0], ri, 0, 0, ccw_send, ccw_recv, ccw_dev)
        zag_wait(ccw_j[1])
        xy_send(ccw_j[1], ri, 0, 1, ccw_send, ccw_recv, ccw_dev)

        for hop in range(1, N_HOP):
            for k in range(2):
                xy_wait(cw_j[k], (ri - hop) % R, hop - 1, k,
                        cw_send, cw_recv, ccw_dev)
                xy_send(cw_j[k], (ri - hop) % R, hop, k,
                        cw_send, cw_recv, cw_dev)
            for k in range(2):
                xy_wait(ccw_j[k], (ri + hop) % R, hop - 1, k,
                        ccw_send, ccw_recv, cw_dev)
                xy_send(ccw_j[k], (ri + hop) % R, hop, k,
                        ccw_send, ccw_recv, ccw_dev)

        for k in range(2):
            xy_wait(cw_j[k], (ri - N_HOP) % R, N_HOP - 1, k,
                    cw_send, cw_recv, ccw_dev)
            xy_wait(ccw_j[k], (ri + N_HOP) % R, N_HOP - 1, k,
                    ccw_send, ccw_recv, cw_dev)

        for rdma in started:
            rdma.wait_send()

    out, _ = pl.pallas_call(
        body,
        out_shape=(
            jax.ShapeDtypeStruct((M, D), jnp.float32),
            jax.ShapeDtypeStruct((Z, SQ, D), jnp.float32),
        ),
        in_specs=[
            pl.BlockSpec(memory_space=pl.ANY),
            pl.BlockSpec(memory_space=pl.ANY),
            pl.BlockSpec(memory_space=pltpu.VMEM),
        ],
        out_specs=(
            pl.BlockSpec(memory_space=pl.ANY),
            pl.BlockSpec(memory_space=pl.ANY),
        ),
        scratch_shapes=[
            pltpu.SemaphoreType.DMA((Z - 1,)),
            pltpu.SemaphoreType.DMA((Z,)),
            pltpu.SemaphoreType.DMA((Z - 1,)),
            pltpu.SemaphoreType.DMA((Z,)),
            pltpu.SemaphoreType.DMA((N_HOP, 2)),
            pltpu.SemaphoreType.DMA((N_HOP, 2)),
            pltpu.SemaphoreType.DMA((N_HOP, 2)),
            pltpu.SemaphoreType.DMA((N_HOP, 2)),
            pltpu.SemaphoreType.DMA((2,)),
            pltpu.VMEM((SQ, D), jnp.float32),
            pltpu.VMEM((SQ, D), jnp.float32),
            pltpu.VMEM((SQ, D), jnp.float32),
            pltpu.VMEM((SQ, D), jnp.float32),
        ],
        compiler_params=pltpu.CompilerParams(collective_id=0),
    )(partial, resid, gamma)
    return out


# device time: 621367 ns/iter; 1.9682x vs baseline; 1.0288x over previous
import jax
import jax.numpy as jnp
from jax import lax
from jax.experimental import pallas as pl
from jax.experimental.pallas import tpu as pltpu

Z = 4
R = 4
M = 4096
D = 4096
Q = M // R
SQ = Q // Z
NS = 2
ST = SQ // NS
EPS = 1e-6
N_HOP = R - 1


def _ring_coords(r):
    xr = r // 2
    return xr, (xr + r % 2) % 2


def kernel(partial, resid, gamma):
    def body(partial_ref, resid_ref, gamma_ref,
             out_ref, rs_hbm,
             rs_send, rs_recv, zag_send, zag_recv,
             cw_send, cw_recv, ccw_send, ccw_recv,
             local_sems, t_a, t_b, t_acc, t_o):
        my_x = lax.axis_index("x")
        my_y = lax.axis_index("y")
        my_z = lax.axis_index("z")
        ri = 2 * my_x + (my_x + my_y) % 2
        cw_dev = _ring_coords((ri + 1) % R) + (my_z,)
        ccw_dev = _ring_coords((ri + 3) % R) + (my_z,)
        row0 = ri * Q + my_z * SQ

        barrier = pltpu.get_barrier_semaphore()
        for d in range(1, Z):
            pl.semaphore_signal(
                barrier, inc=1, device_id=(my_x, my_y, (my_z + d) % Z),
                device_id_type=pl.DeviceIdType.MESH)
        for nbr in (cw_dev, ccw_dev):
            pl.semaphore_signal(barrier, inc=1, device_id=nbr,
                                device_id_type=pl.DeviceIdType.MESH)
        pl.semaphore_wait(barrier, 5)

        started = []

        def load(src, dst, i):
            c = pltpu.make_async_copy(src, dst, local_sems.at[i])
            c.start()
            return c

        for st in range(NS):
            for d in range(1, Z):
                zt = (my_z + d) % Z
                rdma = pltpu.make_async_remote_copy(
                    src_ref=partial_ref.at[
                        0, pl.ds(ri * Q + zt * SQ + st * ST, ST), :],
                    dst_ref=rs_hbm.at[my_z, pl.ds(st * ST, ST), :],
                    send_sem=rs_send.at[st, d - 1],
                    recv_sem=rs_recv.at[st, my_z],
                    device_id=(my_x, my_y, zt),
                    device_id_type=pl.DeviceIdType.MESH,
                )
                rdma.start()
                started.append(rdma)

        for st in range(NS):
            srow0 = row0 + st * ST
            c1 = load(partial_ref.at[0, pl.ds(srow0, ST), :], t_a, 0)
            c2 = load(resid_ref.at[pl.ds(srow0, ST), :], t_b, 1)
            c1.wait()
            c2.wait()
            t_acc[...] = t_a[...] + t_b[...]
            for d in range(1, Z):
                zs = (my_z + d) % Z
                recv = pltpu.make_async_remote_copy(
                    src_ref=partial_ref.at[0, pl.ds(0, ST), :],
                    dst_ref=rs_hbm.at[zs, pl.ds(st * ST, ST), :],
                    send_sem=rs_send.at[st, d - 1],
                    recv_sem=rs_recv.at[st, zs],
                    device_id=(my_x, my_y, zs),
                    device_id_type=pl.DeviceIdType.MESH,
                )
                recv.wait_recv()
                c1 = load(rs_hbm.at[zs, pl.ds(st * ST, ST), :], t_a, 0)
                c1.wait()
                t_acc[...] = t_acc[...] + t_a[...]
            y = t_acc[...]
            rms = jnp.sqrt(jnp.mean(y * y, axis=-1, keepdims=True) + EPS)
            t_o[...] = y / rms * gamma_ref[...][None, :]
            c1 = load(t_o, out_ref.at[pl.ds(srow0, ST), :], 0)
            c1.wait()
            for d in range(1, Z):
                zt = (my_z + d) % Z
                rdma = pltpu.make_async_remote_copy(
                    src_ref=out_ref.at[pl.ds(srow0, ST), :],
                    dst_ref=out_ref.at[pl.ds(srow0, ST), :],
                    send_sem=zag_send.at[st, d - 1],
                    recv_sem=zag_recv.at[st, my_z],
                    device_id=(my_x, my_y, zt),
                    device_id_type=pl.DeviceIdType.MESH,
                )
                rdma.start()
                started.append(rdma)

        def prow(quarter, j, st):
            return quarter * Q + j * SQ + st * ST

        def xy_send(j, quarter, st, hop, k, sems_s, sems_r, dev):
            rdma = pltpu.make_async_remote_copy(
                src_ref=out_ref.at[pl.ds(prow(quarter, j, st), ST), :],
                dst_ref=out_ref.at[pl.ds(prow(quarter, j, st), ST), :],
                send_sem=sems_s.at[st, hop, k],
                recv_sem=sems_r.at[st, hop, k],
                device_id=dev,
                device_id_type=pl.DeviceIdType.MESH,
            )
            rdma.start()
            started.append(rdma)

        def zag_wait(zs, st):
            recv = pltpu.make_async_remote_copy(
                src_ref=out_ref.at[pl.ds(row0, ST), :],
                dst_ref=out_ref.at[pl.ds(prow(ri, zs, st), ST), :],
                send_sem=zag_send.at[st, 0],
                recv_sem=zag_recv.at[st, zs],
                device_id=(my_x, my_y, zs),
                device_id_type=pl.DeviceIdType.MESH,
            )
            recv.wait_recv()

        def xy_wait(j, quarter, st, hop, k, sems_s, sems_r, dev):
            recv = pltpu.make_async_remote_copy(
                src_ref=out_ref.at[pl.ds(row0, ST), :],
                dst_ref=out_ref.at[pl.ds(prow(quarter, j, st), ST), :],
                send_sem=sems_s.at[st, hop, k],
                recv_sem=sems_r.at[st, hop, k],
                device_id=dev,
                device_id_type=pl.DeviceIdType.MESH,
            )
            recv.wait_recv()

        cw_j = [my_z, (my_z + 1) % Z]
        ccw_j = [(my_z + 2) % Z, (my_z + 3) % Z]

        for st in range(NS):
            xy_send(cw_j[0], ri, st, 0, 0, cw_send, cw_recv, cw_dev)
            zag_wait(cw_j[1], st)
            xy_send(cw_j[1], ri, st, 0, 1, cw_send, cw_recv, cw_dev)
            zag_wait(ccw_j[0], st)
            xy_send(ccw_j[0], ri, st, 0, 0, ccw_send, ccw_recv, ccw_dev)
            zag_wait(ccw_j[1], st)
            xy_send(ccw_j[1], ri, st, 0, 1, ccw_send, ccw_recv, ccw_dev)

        for st in range(NS):
            for hop in range(1, N_HOP):
                for k in range(2):
                    xy_wait(cw_j[k], (ri - hop) % R, st, hop - 1, k,
                            cw_send, cw_recv, ccw_dev)
                    xy_send(cw_j[k], (ri - hop) % R, st, hop, k,
                            cw_send, cw_recv, cw_dev)
                for k in range(2):
                    xy_wait(ccw_j[k], (ri + hop) % R, st, hop - 1, k,
                            ccw_send, ccw_recv, cw_dev)
                    xy_send(ccw_j[k], (ri + hop) % R, st, hop, k,
                            ccw_send, ccw_recv, ccw_dev)
            for k in range(2):
                xy_wait(cw_j[k], (ri - N_HOP) % R, st, N_HOP - 1, k,
                        cw_send, cw_recv, ccw_dev)
                xy_wait(ccw_j[k], (ri + N_HOP) % R, st, N_HOP - 1, k,
                        ccw_send, ccw_recv, cw_dev)

        for rdma in started:
            rdma.wait_send()

    out, _ = pl.pallas_call(
        body,
        out_shape=(
            jax.ShapeDtypeStruct((M, D), jnp.float32),
            jax.ShapeDtypeStruct((Z, SQ, D), jnp.float32),
        ),
        in_specs=[
            pl.BlockSpec(memory_space=pl.ANY),
            pl.BlockSpec(memory_space=pl.ANY),
            pl.BlockSpec(memory_space=pltpu.VMEM),
        ],
        out_specs=(
            pl.BlockSpec(memory_space=pl.ANY),
            pl.BlockSpec(memory_space=pl.ANY),
        ),
        scratch_shapes=[
            pltpu.SemaphoreType.DMA((NS, Z - 1)),
            pltpu.SemaphoreType.DMA((NS, Z)),
            pltpu.SemaphoreType.DMA((NS, Z - 1)),
            pltpu.SemaphoreType.DMA((NS, Z)),
            pltpu.SemaphoreType.DMA((NS, N_HOP, 2)),
            pltpu.SemaphoreType.DMA((NS, N_HOP, 2)),
            pltpu.SemaphoreType.DMA((NS, N_HOP, 2)),
            pltpu.SemaphoreType.DMA((NS, N_HOP, 2)),
            pltpu.SemaphoreType.DMA((2,)),
            pltpu.VMEM((ST, D), jnp.float32),
            pltpu.VMEM((ST, D), jnp.float32),
            pltpu.VMEM((ST, D), jnp.float32),
            pltpu.VMEM((ST, D), jnp.float32),
        ],
        compiler_params=pltpu.CompilerParams(collective_id=0),
    )(partial, resid, gamma)
    return out


# device time: 578331 ns/iter; 2.1146x vs baseline; 1.0744x over previous
import jax
import jax.numpy as jnp
from jax import lax
from jax.experimental import pallas as pl
from jax.experimental.pallas import tpu as pltpu

Z = 4
R = 4
M = 4096
D = 4096
Q = M // R
SQ = Q // Z
NS = 2
ST = SQ // NS
EPS = 1e-6
N_HOP = R - 1


def _ring_coords(r):
    xr = r // 2
    return xr, (xr + r % 2) % 2


def kernel(partial, resid, gamma):
    def body(partial_ref, resid_ref, gamma_ref,
             out_ref, rs_hbm,
             rs_send, rs_recv, up_zs, up_zr, dn_zs, dn_zr,
             cw_send, cw_recv, ccw_send, ccw_recv,
             local_sems, t_a, t_b, t_acc, t_o):
        my_x = lax.axis_index("x")
        my_y = lax.axis_index("y")
        my_z = lax.axis_index("z")
        ri = 2 * my_x + (my_x + my_y) % 2
        cw_dev = _ring_coords((ri + 1) % R) + (my_z,)
        ccw_dev = _ring_coords((ri + 3) % R) + (my_z,)
        row0 = ri * Q + my_z * SQ

        barrier = pltpu.get_barrier_semaphore()
        for d in range(1, Z):
            pl.semaphore_signal(
                barrier, inc=1, device_id=(my_x, my_y, (my_z + d) % Z),
                device_id_type=pl.DeviceIdType.MESH)
        for nbr in (cw_dev, ccw_dev):
            pl.semaphore_signal(barrier, inc=1, device_id=nbr,
                                device_id_type=pl.DeviceIdType.MESH)
        pl.semaphore_wait(barrier, 5)

        started = []
        guarded = []

        def load(src, dst, i):
            c = pltpu.make_async_copy(src, dst, local_sems.at[i])
            c.start()
            return c

        def prow(quarter, j, st):
            return quarter * Q + j * SQ + st * ST

        def zag_chain_send(o, st_, up, cond):
            dz = 1 if up else Z - 1
            sems_s = up_zs if up else dn_zs
            sems_r = up_zr if up else dn_zr
            rdma = pltpu.make_async_remote_copy(
                src_ref=out_ref.at[pl.ds(prow(ri, o, st_), ST), :],
                dst_ref=out_ref.at[pl.ds(prow(ri, o, st_), ST), :],
                send_sem=sems_s.at[st_, o],
                recv_sem=sems_r.at[st_, o],
                device_id=(my_x, my_y, (my_z + dz) % Z),
                device_id_type=pl.DeviceIdType.MESH,
            )

            @pl.when(cond)
            def _():
                rdma.start()

            guarded.append((cond, rdma))

        for st in range(NS):
            for d in range(1, Z):
                zt = (my_z + d) % Z
                rdma = pltpu.make_async_remote_copy(
                    src_ref=partial_ref.at[
                        0, pl.ds(ri * Q + zt * SQ + st * ST, ST), :],
                    dst_ref=rs_hbm.at[my_z, pl.ds(st * ST, ST), :],
                    send_sem=rs_send.at[st, d - 1],
                    recv_sem=rs_recv.at[st, my_z],
                    device_id=(my_x, my_y, zt),
                    device_id_type=pl.DeviceIdType.MESH,
                )
                rdma.start()
                started.append(rdma)

        for st in range(NS):
            srow0 = row0 + st * ST
            c1 = load(partial_ref.at[0, pl.ds(srow0, ST), :], t_a, 0)
            c2 = load(resid_ref.at[pl.ds(srow0, ST), :], t_b, 1)
            c1.wait()
            c2.wait()
            t_acc[...] = t_a[...] + t_b[...]
            for d in range(1, Z):
                zs = (my_z + d) % Z
                recv = pltpu.make_async_remote_copy(
                    src_ref=partial_ref.at[0, pl.ds(0, ST), :],
                    dst_ref=rs_hbm.at[zs, pl.ds(st * ST, ST), :],
                    send_sem=rs_send.at[st, d - 1],
                    recv_sem=rs_recv.at[st, zs],
                    device_id=(my_x, my_y, zs),
                    device_id_type=pl.DeviceIdType.MESH,
                )
                recv.wait_recv()
                c1 = load(rs_hbm.at[zs, pl.ds(st * ST, ST), :], t_a, 0)
                c1.wait()
                t_acc[...] = t_acc[...] + t_a[...]
            y = t_acc[...]
            rms = jnp.sqrt(jnp.mean(y * y, axis=-1, keepdims=True) + EPS)
            t_o[...] = y / rms * gamma_ref[...][None, :]
            c1 = load(t_o, out_ref.at[pl.ds(srow0, ST), :], 0)
            c1.wait()
            zag_chain_send(my_z, st, True, my_z < Z - 1)
            zag_chain_send(my_z, st, False, my_z > 0)

        def xy_send(j, quarter, st, hop, k, sems_s, sems_r, dev):
            rdma = pltpu.make_async_remote_copy(
                src_ref=out_ref.at[pl.ds(prow(quarter, j, st), ST), :],
                dst_ref=out_ref.at[pl.ds(prow(quarter, j, st), ST), :],
                send_sem=sems_s.at[st, hop, k],
                recv_sem=sems_r.at[st, hop, k],
                device_id=dev,
                device_id_type=pl.DeviceIdType.MESH,
            )
            rdma.start()
            started.append(rdma)

        def xy_wait(j, quarter, st, hop, k, sems_s, sems_r, dev):
            recv = pltpu.make_async_remote_copy(
                src_ref=out_ref.at[pl.ds(row0, ST), :],
                dst_ref=out_ref.at[pl.ds(prow(quarter, j, st), ST), :],
                send_sem=sems_s.at[st, hop, k],
                recv_sem=sems_r.at[st, hop, k],
                device_id=dev,
                device_id_type=pl.DeviceIdType.MESH,
            )
            recv.wait_recv()

        cw_j = [my_z, (my_z + 1) % Z]
        ccw_j = [(my_z + 2) % Z, (my_z + 3) % Z]

        def sel(a, b, c, d_):
            return jnp.where(my_z == 0, a, jnp.where(
                my_z == 1, b, jnp.where(my_z == 2, c, d_)))

        o_order = [sel(1, 0, 1, 2), sel(2, 2, 3, 1), sel(3, 3, 0, 0)]

        def handle_piece(o, st):
            up_recv = pltpu.make_async_remote_copy(
                src_ref=out_ref.at[pl.ds(row0, ST), :],
                dst_ref=out_ref.at[pl.ds(prow(ri, o, st), ST), :],
                send_sem=up_zs.at[st, o],
                recv_sem=up_zr.at[st, o],
                device_id=(my_x, my_y, my_z),
                device_id_type=pl.DeviceIdType.MESH,
            )
            dn_recv = pltpu.make_async_remote_copy(
                src_ref=out_ref.at[pl.ds(row0, ST), :],
                dst_ref=out_ref.at[pl.ds(prow(ri, o, st), ST), :],
                send_sem=dn_zs.at[st, o],
                recv_sem=dn_zr.at[st, o],
                device_id=(my_x, my_y, my_z),
                device_id_type=pl.DeviceIdType.MESH,
            )

            @pl.when(o < my_z)
            def _():
                up_recv.wait_recv()

            @pl.when(o > my_z)
            def _():
                dn_recv.wait_recv()

            zag_chain_send(o, st, True, (o < my_z) & (my_z < Z - 1))
            zag_chain_send(o, st, False, (o > my_z) & (my_z > 0))

            for cond, k, ss, sr, dv in (
                ((o == cw_j[1]), 1, cw_send, cw_recv, cw_dev),
                ((o == ccw_j[0]), 0, ccw_send, ccw_recv, ccw_dev),
                ((o == ccw_j[1]), 1, ccw_send, ccw_recv, ccw_dev),
            ):
                rdma = pltpu.make_async_remote_copy(
                    src_ref=out_ref.at[pl.ds(prow(ri, o, st), ST), :],
                    dst_ref=out_ref.at[pl.ds(prow(ri, o, st), ST), :],
                    send_sem=ss.at[st, 0, k],
                    recv_sem=sr.at[st, 0, k],
                    device_id=dv,
                    device_id_type=pl.DeviceIdType.MESH,
                )

                @pl.when(cond)
                def _(rdma=rdma):
                    rdma.start()

                guarded.append((cond, rdma))

        for st in range(NS):
            xy_send(cw_j[0], ri, st, 0, 0, cw_send, cw_recv, cw_dev)
            for o in o_order:
                handle_piece(o, st)

        for st in range(NS):
            for hop in range(1, N_HOP):
                for k in range(2):
                    xy_wait(cw_j[k], (ri - hop) % R, st, hop - 1, k,
                            cw_send, cw_recv, ccw_dev)
                    xy_send(cw_j[k], (ri - hop) % R, st, hop, k,
                            cw_send, cw_recv, cw_dev)
                for k in range(2):
                    xy_wait(ccw_j[k], (ri + hop) % R, st, hop - 1, k,
                            ccw_send, ccw_recv, cw_dev)
                    xy_send(ccw_j[k], (ri + hop) % R, st, hop, k,
                            ccw_send, ccw_recv, ccw_dev)
            for k in range(2):
                xy_wait(cw_j[k], (ri - N_HOP) % R, st, N_HOP - 1, k,
                        cw_send, cw_recv, ccw_dev)
                xy_wait(ccw_j[k], (ri + N_HOP) % R, st, N_HOP - 1, k,
                        ccw_send, ccw_recv, cw_dev)

        for rdma in started:
            rdma.wait_send()
        for cond, rdma in guarded:
            @pl.when(cond)
            def _(rdma=rdma):
                rdma.wait_send()

    out, _ = pl.pallas_call(
        body,
        out_shape=(
            jax.ShapeDtypeStruct((M, D), jnp.float32),
            jax.ShapeDtypeStruct((Z, SQ, D), jnp.float32),
        ),
        in_specs=[
            pl.BlockSpec(memory_space=pl.ANY),
            pl.BlockSpec(memory_space=pl.ANY),
            pl.BlockSpec(memory_space=pltpu.VMEM),
        ],
        out_specs=(
            pl.BlockSpec(memory_space=pl.ANY),
            pl.BlockSpec(memory_space=pl.ANY),
        ),
        scratch_shapes=[
            pltpu.SemaphoreType.DMA((NS, Z - 1)),
            pltpu.SemaphoreType.DMA((NS, Z)),
            pltpu.SemaphoreType.DMA((NS, Z)),
            pltpu.SemaphoreType.DMA((NS, Z)),
            pltpu.SemaphoreType.DMA((NS, Z)),
            pltpu.SemaphoreType.DMA((NS, Z)),
            pltpu.SemaphoreType.DMA((NS, N_HOP, 2)),
            pltpu.SemaphoreType.DMA((NS, N_HOP, 2)),
            pltpu.SemaphoreType.DMA((NS, N_HOP, 2)),
            pltpu.SemaphoreType.DMA((NS, N_HOP, 2)),
            pltpu.SemaphoreType.DMA((2,)),
            pltpu.VMEM((ST, D), jnp.float32),
            pltpu.VMEM((ST, D), jnp.float32),
            pltpu.VMEM((ST, D), jnp.float32),
            pltpu.VMEM((ST, D), jnp.float32),
        ],
        compiler_params=pltpu.CompilerParams(collective_id=0),
    )(partial, resid, gamma)
    return out


# device time: 529537 ns/iter; 2.3095x vs baseline; 1.0921x over previous
import jax
import jax.numpy as jnp
from jax import lax
from jax.experimental import pallas as pl
from jax.experimental.pallas import tpu as pltpu

Z = 4
R = 4
M = 4096
D = 4096
Q = M // R
SQ = Q // Z
NS = 2
ST = SQ // NS
EPS = 1e-6
N_HOP = R - 1


def _ring_coords(r):
    xr = r // 2
    return xr, (xr + r % 2) % 2


def kernel(partial, resid, gamma):
    def body(partial_ref, resid_ref, gamma_ref,
             out_ref, rs_hbm, fwd_hbm,
             rs_send, rs_recv, up_zs, up_zr, dn_zs, dn_zr,
             cw_send, cw_recv, ccw_send, ccw_recv,
             local_sems, t_a, t_b, t_acc, t_o):
        my_x = lax.axis_index("x")
        my_y = lax.axis_index("y")
        my_z = lax.axis_index("z")
        ri = 2 * my_x + (my_x + my_y) % 2
        cw_dev = _ring_coords((ri + 1) % R) + (my_z,)
        ccw_dev = _ring_coords((ri + 3) % R) + (my_z,)
        row0 = ri * Q + my_z * SQ

        barrier = pltpu.get_barrier_semaphore()
        for d in range(1, Z):
            pl.semaphore_signal(
                barrier, inc=1, device_id=(my_x, my_y, (my_z + d) % Z),
                device_id_type=pl.DeviceIdType.MESH)
        for nbr in (cw_dev, ccw_dev):
            pl.semaphore_signal(barrier, inc=1, device_id=nbr,
                                device_id_type=pl.DeviceIdType.MESH)
        pl.semaphore_wait(barrier, 5)

        started = []
        guarded = []

        def load(src, dst, i):
            c = pltpu.make_async_copy(src, dst, local_sems.at[i])
            c.start()
            return c

        def prow(quarter, j, st):
            return quarter * Q + j * SQ + st * ST

        def zag_chain_send(o, st_, up, cond):
            dz = 1 if up else Z - 1
            sems_s = up_zs if up else dn_zs
            sems_r = up_zr if up else dn_zr
            rdma = pltpu.make_async_remote_copy(
                src_ref=out_ref.at[pl.ds(prow(ri, o, st_), ST), :],
                dst_ref=out_ref.at[pl.ds(prow(ri, o, st_), ST), :],
                send_sem=sems_s.at[st_, o],
                recv_sem=sems_r.at[st_, o],
                device_id=(my_x, my_y, (my_z + dz) % Z),
                device_id_type=pl.DeviceIdType.MESH,
            )

            @pl.when(cond)
            def _():
                rdma.start()

            guarded.append((cond, rdma))

        def psrc(j, st):
            return partial_ref.at[0, pl.ds(prow(ri, j, st), ST), :]

        def rs2(i, st, src, slot, zt, cond):
            rdma = pltpu.make_async_remote_copy(
                src_ref=src,
                dst_ref=rs_hbm.at[slot, pl.ds(st * ST, ST), :],
                send_sem=rs_send.at[st, i],
                recv_sem=rs_recv.at[st, slot],
                device_id=(my_x, my_y, zt),
                device_id_type=pl.DeviceIdType.MESH,
            )

            @pl.when(cond)
            def _():
                rdma.start()

            guarded.append((cond, rdma))

        def rs2_wait(slot, st):
            recv = pltpu.make_async_remote_copy(
                src_ref=psrc(0, st),
                dst_ref=rs_hbm.at[slot, pl.ds(st * ST, ST), :],
                send_sem=rs_send.at[st, 0],
                recv_sem=rs_recv.at[st, slot],
                device_id=(my_x, my_y, my_z),
                device_id_type=pl.DeviceIdType.MESH,
            )
            recv.wait_recv()

        for st in range(NS):
            rs2(0, st, psrc(3, st), 3, 1, my_z == 0)
            rs2(1, st, psrc(2, st), 2, 1, my_z == 0)
            rs2(2, st, psrc(1, st), 0, 1, my_z == 0)
            rs2(0, st, psrc(0, st), 0, 0, my_z == 1)
            rs2(0, st, psrc(3, st), 0, 3, my_z == 2)
            rs2(0, st, psrc(0, st), 3, 2, my_z == 3)
            rs2(1, st, psrc(1, st), 2, 2, my_z == 3)
            rs2(2, st, psrc(2, st), 0, 2, my_z == 3)

        def combine(st, in_slot, j, fwd_slot, send_i, zt, cond):
            send = pltpu.make_async_remote_copy(
                src_ref=fwd_hbm.at[fwd_slot, pl.ds(st * ST, ST), :],
                dst_ref=rs_hbm.at[1, pl.ds(st * ST, ST), :],
                send_sem=rs_send.at[st, send_i],
                recv_sem=rs_recv.at[st, 1],
                device_id=(my_x, my_y, zt),
                device_id_type=pl.DeviceIdType.MESH,
            )

            @pl.when(cond)
            def _():
                rs2_wait(in_slot, st)
                c1 = load(rs_hbm.at[in_slot, pl.ds(st * ST, ST), :], t_a, 0)
                c2 = load(psrc(j, st), t_b, 1)
                c1.wait()
                c2.wait()
                t_o[...] = t_a[...] + t_b[...]
                c1 = load(t_o, fwd_hbm.at[fwd_slot, pl.ds(st * ST, ST), :], 0)
                c1.wait()
                send.start()

            guarded.append((cond, send))

        for st in range(NS):
            combine(st, 3, 3, 0, 1, 3, my_z == 1)
            combine(st, 2, 2, 1, 2, 2, my_z == 1)
            combine(st, 3, 0, 0, 1, 0, my_z == 2)
            combine(st, 2, 1, 1, 2, 1, my_z == 2)

            srow0 = row0 + st * ST
            c1 = load(partial_ref.at[0, pl.ds(srow0, ST), :], t_a, 0)
            c2 = load(resid_ref.at[pl.ds(srow0, ST), :], t_b, 1)
            c1.wait()
            c2.wait()
            t_acc[...] = t_a[...] + t_b[...]
            for slot in (0, 1):
                rs2_wait(slot, st)
                c1 = load(rs_hbm.at[slot, pl.ds(st * ST, ST), :], t_a, 0)
                c1.wait()
                t_acc[...] = t_acc[...] + t_a[...]
            y = t_acc[...]
            rms = jnp.sqrt(jnp.mean(y * y, axis=-1, keepdims=True) + EPS)
            t_o[...] = y / rms * gamma_ref[...][None, :]
            c1 = load(t_o, out_ref.at[pl.ds(srow0, ST), :], 0)
            c1.wait()
            zag_chain_send(my_z, st, True, my_z < Z - 1)
            zag_chain_send(my_z, st, False, my_z > 0)

        def xy_send(j, quarter, st, hop, k, sems_s, sems_r, dev):
            rdma = pltpu.make_async_remote_copy(
                src_ref=out_ref.at[pl.ds(prow(quarter, j, st), ST), :],
                dst_ref=out_ref.at[pl.ds(prow(quarter, j, st), ST), :],
                send_sem=sems_s.at[st, hop, k],
                recv_sem=sems_r.at[st, hop, k],
                device_id=dev,
                device_id_type=pl.DeviceIdType.MESH,
            )
            rdma.start()
            started.append(rdma)

        def xy_wait(j, quarter, st, hop, k, sems_s, sems_r, dev):
            recv = pltpu.make_async_remote_copy(
                src_ref=out_ref.at[pl.ds(row0, ST), :],
                dst_ref=out_ref.at[pl.ds(prow(quarter, j, st), ST), :],
                send_sem=sems_s.at[st, hop, k],
                recv_sem=sems_r.at[st, hop, k],
                device_id=dev,
                device_id_type=pl.DeviceIdType.MESH,
            )
            recv.wait_recv()

        cw_j = [my_z, (my_z + 1) % Z]
        ccw_j = [(my_z + 2) % Z, (my_z + 3) % Z]

        def sel(a, b, c, d_):
            return jnp.where(my_z == 0, a, jnp.where(
                my_z == 1, b, jnp.where(my_z == 2, c, d_)))

        o_order = [sel(1, 0, 1, 2), sel(2, 2, 3, 1), sel(3, 3, 0, 0)]

        def handle_piece(o, st):
            up_recv = pltpu.make_async_remote_copy(
                src_ref=out_ref.at[pl.ds(row0, ST), :],
                dst_ref=out_ref.at[pl.ds(prow(ri, o, st), ST), :],
                send_sem=up_zs.at[st, o],
                recv_sem=up_zr.at[st, o],
                device_id=(my_x, my_y, my_z),
                device_id_type=pl.DeviceIdType.MESH,
            )
            dn_recv = pltpu.make_async_remote_copy(
                src_ref=out_ref.at[pl.ds(row0, ST), :],
                dst_ref=out_ref.at[pl.ds(prow(ri, o, st), ST), :],
                send_sem=dn_zs.at[st, o],
                recv_sem=dn_zr.at[st, o],
                device_id=(my_x, my_y, my_z),
                device_id_type=pl.DeviceIdType.MESH,
            )

            @pl.when(o < my_z)
            def _():
                up_recv.wait_recv()

            @pl.when(o > my_z)
            def _():
                dn_recv.wait_recv()

            zag_chain_send(o, st, True, (o < my_z) & (my_z < Z - 1))
            zag_chain_send(o, st, False, (o > my_z) & (my_z > 0))

            for cond, k, ss, sr, dv in (
                ((o == cw_j[1]), 1, cw_send, cw_recv, cw_dev),
                ((o == ccw_j[0]), 0, ccw_send, ccw_recv, ccw_dev),
                ((o == ccw_j[1]), 1, ccw_send, ccw_recv, ccw_dev),
            ):
                rdma = pltpu.make_async_remote_copy(
                    src_ref=out_ref.at[pl.ds(prow(ri, o, st), ST), :],
                    dst_ref=out_ref.at[pl.ds(prow(ri, o, st), ST), :],
                    send_sem=ss.at[st, 0, k],
                    recv_sem=sr.at[st, 0, k],
                    device_id=dv,
                    device_id_type=pl.DeviceIdType.MESH,
                )

                @pl.when(cond)
                def _(rdma=rdma):
                    rdma.start()

                guarded.append((cond, rdma))

        for st in range(NS):
            xy_send(cw_j[0], ri, st, 0, 0, cw_send, cw_recv, cw_dev)
            for o in o_order:
                handle_piece(o, st)

        for st in range(NS):
            for hop in range(1, N_HOP):
                for k in range(2):
                    xy_wait(cw_j[k], (ri - hop) % R, st, hop - 1, k,
                            cw_send, cw_recv, ccw_dev)
                    xy_send(cw_j[k], (ri - hop) % R, st, hop, k,
                            cw_send, cw_recv, cw_dev)
                for k in range(2):
                    xy_wait(ccw_j[k], (ri + hop) % R, st, hop - 1, k,
                            ccw_send, ccw_recv, cw_dev)
                    xy_send(ccw_j[k], (ri + hop) % R, st, hop, k,
                            ccw_send, ccw_recv, ccw_dev)
            for k in range(2):
                xy_wait(cw_j[k], (ri - N_HOP) % R, st, N_HOP - 1, k,
                        cw_send, cw_recv, ccw_dev)
                xy_wait(ccw_j[k], (ri + N_HOP) % R, st, N_HOP - 1, k,
                        ccw_send, ccw_recv, cw_dev)

        for rdma in started:
            rdma.wait_send()
        for cond, rdma in guarded:
            @pl.when(cond)
            def _(rdma=rdma):
                rdma.wait_send()

    out, _, _ = pl.pallas_call(
        body,
        out_shape=(
            jax.ShapeDtypeStruct((M, D), jnp.float32),
            jax.ShapeDtypeStruct((Z, SQ, D), jnp.float32),
            jax.ShapeDtypeStruct((2, SQ, D), jnp.float32),
        ),
        in_specs=[
            pl.BlockSpec(memory_space=pl.ANY),
            pl.BlockSpec(memory_space=pl.ANY),
            pl.BlockSpec(memory_space=pltpu.VMEM),
        ],
        out_specs=(
            pl.BlockSpec(memory_space=pl.ANY),
            pl.BlockSpec(memory_space=pl.ANY),
            pl.BlockSpec(memory_space=pl.ANY),
        ),
        scratch_shapes=[
            pltpu.SemaphoreType.DMA((NS, Z - 1)),
            pltpu.SemaphoreType.DMA((NS, Z)),
            pltpu.SemaphoreType.DMA((NS, Z)),
            pltpu.SemaphoreType.DMA((NS, Z)),
            pltpu.SemaphoreType.DMA((NS, Z)),
            pltpu.SemaphoreType.DMA((NS, Z)),
            pltpu.SemaphoreType.DMA((NS, N_HOP, 2)),
            pltpu.SemaphoreType.DMA((NS, N_HOP, 2)),
            pltpu.SemaphoreType.DMA((NS, N_HOP, 2)),
            pltpu.SemaphoreType.DMA((NS, N_HOP, 2)),
            pltpu.SemaphoreType.DMA((2,)),
            pltpu.VMEM((ST, D), jnp.float32),
            pltpu.VMEM((ST, D), jnp.float32),
            pltpu.VMEM((ST, D), jnp.float32),
            pltpu.VMEM((ST, D), jnp.float32),
        ],
        compiler_params=pltpu.CompilerParams(collective_id=0),
    )(partial, resid, gamma)
    return out
